# baseline (speedup 1.0000x reference)
import functools

import jax
import jax.numpy as jnp
from jax import lax
from jax.experimental import pallas as pl
from jax.experimental.pallas import tpu as pltpu
from jax.experimental.pallas import tpu_sc as plsc

N = 100000
NP = 102400
E = 3200000
G = 1000
GP = 1024
DH = 64
NC, NS = 2, 16
NW = NC * NS
WIN = 128
J = 4
NWIN = 25088
EP = NWIN * WIN
ROWB = 4096
NSUB = NP // NW
NZCH = NP // NS

_mesh = functools.partial(
    plsc.VectorSubcoreMesh,
    core_axis_name="c", subcore_axis_name="s", num_cores=NC, num_subcores=NS)

_SC_PARAMS = pltpu.CompilerParams(use_tc_tiling_on_sc=False)


def _sc_deg(dst2d):
  @functools.partial(
      pl.kernel,
      out_type=jax.ShapeDtypeStruct((2 * NP,), jnp.float32),
      mesh=_mesh(),
      compiler_params=_SC_PARAMS,
      scratch_types=[
          pltpu.VMEM_SHARED((NP,), jnp.float32),
          pltpu.VMEM((2, J, WIN), jnp.int32),
          pltpu.VMEM((WIN,), jnp.float32),
          pltpu.VMEM((NZCH,), jnp.float32),
          pltpu.SemaphoreType.DMA,
      ])
  def k(dst_hbm, out_hbm, acc, iv, ones, zb, sem_i):
    c = lax.axis_index("c")
    s = lax.axis_index("s")

    @pl.loop(0, NZCH // 16)
    def _(i):
      zb[pl.ds(i * 16, 16)] = jnp.zeros((16,), jnp.float32)

    @pl.loop(0, WIN // 16)
    def _(i):
      ones[pl.ds(i * 16, 16)] = jnp.ones((16,), jnp.float32)

    pltpu.sync_copy(zb, acc.at[pl.ds(s * NZCH, NZCH)])
    plsc.subcore_barrier()

    per_sub = NWIN // 2 // NS
    base = (c * NS + s) * per_sub
    nb = per_sub // J

    def d_start(b, batch):
      pltpu.async_copy(dst_hbm.at[pl.ds(base + batch * J, J)], iv.at[b],
                       sem_i)

    def d_wait(b):
      pltpu.make_async_copy(dst_hbm.at[pl.ds(base, J)], iv.at[b],
                            sem_i).wait()

    d_start(0, 0)

    @pl.loop(0, nb // 2)
    def _(tp):
      for b in range(2):
        batch = 2 * tp + b
        d_wait(b)
        if b == 0:
          d_start(1, batch + 1)
        else:
          @pl.when(tp < nb // 2 - 1)
          def _():
            d_start(0, batch + 1)
        for j in range(J):
          pltpu.sync_copy(ones, acc.at[iv.at[b].at[j]], add=True)

    plsc.subcore_barrier()
    pltpu.sync_copy(acc.at[pl.ds(s * NZCH, NZCH)],
                    out_hbm.at[pl.ds(c * NP + s * NZCH, NZCH)])

  return k(dst2d)


def _sc_agg(tstk, src2d, dst2d, zrows):
  ot = jax.ShapeDtypeStruct((4 * NP, 16), jnp.float32)

  @functools.partial(
      pl.kernel,
      out_type=ot,
      mesh=_mesh(),
      compiler_params=_SC_PARAMS,
      scratch_types=[
          pltpu.VMEM_SHARED((NP, 16), jnp.float32),
          pltpu.VMEM((2, J, WIN), jnp.int32),
          pltpu.VMEM((4, J, WIN), jnp.int32),
          pltpu.VMEM((2, J, WIN, 16), jnp.float32),
          pltpu.SemaphoreType.DMA,
          pltpu.SemaphoreType.DMA,
          pltpu.SemaphoreType.DMA,
      ])
  def k(t_h, src_h, dst_h, zr, o_h,
        acc, siv, div, rows, sem_i, sem_g, sem_s):
    c = lax.axis_index("c")
    s = lax.axis_index("s")

    per_sub = NWIN // NS
    nbatch = per_sub // J
    nq = nbatch // 4

    def process(p):
      table = t_h.at[pl.ds(p * NP, NP)]
      pltpu.sync_copy(zr, acc.at[pl.ds(s * NZCH, NZCH)])
      plsc.subcore_barrier()

      base = s * per_sub

      def start_idx(bs, bd, batch):
        w0 = base + batch * J
        pltpu.async_copy(src_h.at[pl.ds(w0, J)], siv.at[bs], sem_i)
        pltpu.async_copy(dst_h.at[pl.ds(w0, J)], div.at[bd], sem_i)

      def wait_idx(bs, bd):
        pltpu.make_async_copy(src_h.at[pl.ds(base, J)], siv.at[bs],
                              sem_i).wait()
        pltpu.make_async_copy(dst_h.at[pl.ds(base, J)], div.at[bd],
                              sem_i).wait()

      def drain_scatters(br, bd):
        for j in range(J):
          pltpu.make_async_copy(rows.at[br].at[j],
                                acc.at[div.at[bd].at[j]], sem_s).wait()

      def phase(q, tq):
        batch = 4 * tq + q
        br = q % 2
        if q >= 2:
          drain_scatters(br, q - 2)
        else:
          @pl.when(tq > 0)
          def _():
            drain_scatters(br, q + 2)
        wait_idx(br, q)
        nxt_s, nxt_d = (q + 1) % 2, (q + 1) % 4
        if q < 3:
          start_idx(nxt_s, nxt_d, batch + 1)
        else:
          @pl.when(tq < nq - 1)
          def _():
            start_idx(nxt_s, nxt_d, batch + 1)
        descs = [pltpu.async_copy(table.at[siv.at[br].at[j]],
                                  rows.at[br].at[j], sem_g)
                 for j in range(J)]
        for d in descs:
          d.wait()
        for j in range(J):
          pltpu.async_copy(rows.at[br].at[j], acc.at[div.at[q].at[j]],
                           sem_s, add=True)

      start_idx(0, 0, 0)

      @pl.loop(0, nq)
      def _(tq):
        for q in range(4):
          phase(q, tq)

      drain_scatters(0, 2)
      drain_scatters(1, 3)

      plsc.subcore_barrier()
      pltpu.sync_copy(acc.at[pl.ds(s * NZCH, NZCH)],
                      o_h.at[pl.ds(p * NP + s * NZCH, NZCH)])
      plsc.subcore_barrier()

    @pl.when(c == 0)
    def _():
      process(0)
      process(1)

    @pl.when(c == 1)
    def _():
      process(2)
      process(3)

  return k(tstk, src2d, dst2d, zrows)


def _sc_segmax(h2p, batchp):
  @functools.partial(
      pl.kernel,
      out_type=jax.ShapeDtypeStruct((NW, GP * DH), jnp.float32),
      mesh=_mesh(),
      compiler_params=_SC_PARAMS,
      scratch_types=[
          pltpu.VMEM((GP * DH,), jnp.float32),
          pltpu.VMEM((320 * DH,), jnp.float32),
          pltpu.VMEM((NSUB,), jnp.int32),
      ])
  def k(h_h, b_h, o_h, accv, hv, bv):
    c = lax.axis_index("c")
    s = lax.axis_index("s")
    wid = s * NC + c

    @pl.loop(0, GP * DH // 16)
    def _(i):
      accv[pl.ds(i * 16, 16)] = jnp.full((16,), -jnp.inf, jnp.float32)

    pltpu.sync_copy(b_h.at[pl.ds(wid * NSUB, NSUB)], bv)

    ninf = jnp.full((16,), -jnp.inf, jnp.float32)

    def outer(t, carry):
      pltpu.sync_copy(h_h.at[pl.ds((wid * NSUB + t * 320) * DH, 320 * DH)],
                      hv)

      def inner(gi, cy):
        bvec = bv[pl.ds(t * 320 + gi * 16, 16)]
        for rr in range(16):
          cur, m0, m1, m2, m3 = cy
          r = gi * 16 + rr
          b = bvec[rr]
          r0 = hv[pl.ds(r * DH, 16)]
          r1 = hv[pl.ds(r * DH + 16, 16)]
          r2 = hv[pl.ds(r * DH + 32, 16)]
          r3 = hv[pl.ds(r * DH + 48, 16)]
          ch = b != cur

          @pl.when(ch)
          def _():
            accv[pl.ds(cur * DH, 16)] = m0
            accv[pl.ds(cur * DH + 16, 16)] = m1
            accv[pl.ds(cur * DH + 32, 16)] = m2
            accv[pl.ds(cur * DH + 48, 16)] = m3

          m0n = jnp.where(ch, r0, jnp.maximum(m0, r0))
          m1n = jnp.where(ch, r1, jnp.maximum(m1, r1))
          m2n = jnp.where(ch, r2, jnp.maximum(m2, r2))
          m3n = jnp.where(ch, r3, jnp.maximum(m3, r3))
          cy = (jnp.where(ch, b, cur), m0n, m1n, m2n, m3n)
        return cy

      return lax.fori_loop(0, 20, inner, carry)

    cur0 = bv[pl.ds(0, 16)][0]
    cur, m0, m1, m2, m3 = lax.fori_loop(
        0, NSUB // 320, outer, (cur0, ninf, ninf, ninf, ninf))
    accv[pl.ds(cur * DH, 16)] = m0
    accv[pl.ds(cur * DH + 16, 16)] = m1
    accv[pl.ds(cur * DH + 32, 16)] = m2
    accv[pl.ds(cur * DH + 48, 16)] = m3
    pltpu.sync_copy(accv, o_h.at[wid])

  return k(h2p, batchp)


def _tc_prep(xp, W1, degp):
  nb = NP // ROWB

  def body(xr, wr, wsr, d0r, d1r, o, ot, od):
    dv1 = lax.rsqrt(d0r[...] + d1r[...] + 1.0)
    od[...] = dv1
    h = jnp.dot(xr[...], wr[...], preferred_element_type=jnp.float32)
    o[...] = h * dv1[:, None]
    hs = jnp.dot(xr[...], wsr[0], preferred_element_type=jnp.float32)
    ot[...] = hs * dv1[:, None]

  return pl.pallas_call(
      body, grid=(nb, 4),
      in_specs=[pl.BlockSpec((ROWB, 16), lambda i, p: (i, 0)),
                pl.BlockSpec((16, DH), lambda i, p: (0, 0)),
                pl.BlockSpec((1, 16, 16), lambda i, p: (p, 0, 0)),
                pl.BlockSpec((ROWB,), lambda i, p: (i,)),
                pl.BlockSpec((ROWB,), lambda i, p: (i + nb,))],
      out_specs=[pl.BlockSpec((ROWB, DH), lambda i, p: (i, 0)),
                 pl.BlockSpec((ROWB, 16), lambda i, p: (p * nb + i, 0)),
                 pl.BlockSpec((ROWB,), lambda i, p: (i,))],
      out_shape=[jax.ShapeDtypeStruct((NP, DH), jnp.float32),
                 jax.ShapeDtypeStruct((4 * NP, 16), jnp.float32),
                 jax.ShapeDtypeStruct((NP,), jnp.float32)])(
          xp, W1,
          jnp.stack([W1[:, 16 * p:16 * (p + 1)] for p in range(4)]),
          degp, degp)


def _tc_mid(sstk, a, dinv, b, W):
  nb = NP // ROWB

  def body(s0r, s1r, s2r, s3r, ar, dr, br, wr, wsr, o, ot):
    sf = jnp.concatenate([s0r[...], s1r[...], s2r[...], s3r[...]], axis=1)
    dv = dr[...][:, None]
    x2 = jnp.maximum((sf + ar[...]) * dv + br[...][None, :], 0.0)
    o[...] = jnp.dot(x2, wr[...], preferred_element_type=jnp.float32) * dv
    ot[...] = jnp.dot(x2, wsr[0], preferred_element_type=jnp.float32) * dv

  return pl.pallas_call(
      body, grid=(nb, 4),
      in_specs=[pl.BlockSpec((ROWB, 16), lambda i, p, q=q: (q * nb + i, 0))
                for q in range(4)] +
               [pl.BlockSpec((ROWB, DH), lambda i, p: (i, 0)),
                pl.BlockSpec((ROWB,), lambda i, p: (i,)),
                pl.BlockSpec((DH,), lambda i, p: (0,)),
                pl.BlockSpec((DH, DH), lambda i, p: (0, 0)),
                pl.BlockSpec((1, DH, 16), lambda i, p: (p, 0, 0))],
      out_specs=[pl.BlockSpec((ROWB, DH), lambda i, p: (i, 0)),
                 pl.BlockSpec((ROWB, 16), lambda i, p: (p * nb + i, 0))],
      out_shape=[jax.ShapeDtypeStruct((NP, DH), jnp.float32),
                 jax.ShapeDtypeStruct((4 * NP, 16), jnp.float32)])(
          sstk, sstk, sstk, sstk, a, dinv, b, W,
          jnp.stack([W[:, 16 * p:16 * (p + 1)] for p in range(4)]))


def _tc_post(sstk, a, dinv, b):
  nb = NP // ROWB

  def body(s0r, s1r, s2r, s3r, ar, dr, br, o):
    sf = jnp.concatenate([s0r[...], s1r[...], s2r[...], s3r[...]], axis=1)
    dv = dr[...][:, None]
    o[...] = jnp.maximum((sf + ar[...]) * dv + br[...][None, :], 0.0)

  return pl.pallas_call(
      body, grid=(nb,),
      in_specs=[pl.BlockSpec((ROWB, 16), lambda i, q=q: (q * nb + i, 0))
                for q in range(4)] +
               [pl.BlockSpec((ROWB, DH), lambda i: (i, 0)),
                pl.BlockSpec((ROWB,), lambda i: (i,)),
                pl.BlockSpec((DH,), lambda i: (0,))],
      out_specs=pl.BlockSpec((ROWB, DH), lambda i: (i, 0)),
      out_shape=jax.ShapeDtypeStruct((NP, DH), jnp.float32))(
          sstk, sstk, sstk, sstk, a, dinv, b)


def _tc_head(pm, Wc1, bc1, Wc2, bc2, Wo, bo):
  def body(pr, w1r, b1r, w2r, b2r, wor, bor, o):
    g = jnp.max(pr[...], axis=0)[:G]
    g = jnp.maximum(
        jnp.dot(g, w1r[...], preferred_element_type=jnp.float32)
        + b1r[...][None, :], 0.0)
    g = jnp.maximum(
        jnp.dot(g, w2r[...], preferred_element_type=jnp.float32)
        + b2r[...][None, :], 0.0)
    z = (jnp.dot(g, wor[...], preferred_element_type=jnp.float32)
         + bor[...][None, :])
    o[...] = 1.0 / (1.0 + jnp.exp(-z))

  return pl.pallas_call(
      body,
      in_specs=[pl.BlockSpec((NW, GP, DH), lambda: (0, 0, 0)),
                pl.BlockSpec((DH, DH), lambda: (0, 0)),
                pl.BlockSpec((DH,), lambda: (0,)),
                pl.BlockSpec((DH, DH), lambda: (0, 0)),
                pl.BlockSpec((DH,), lambda: (0,)),
                pl.BlockSpec((DH, 1), lambda: (0, 0)),
                pl.BlockSpec((1,), lambda: (0,))],
      out_specs=pl.BlockSpec((G, 1), lambda: (0, 0)),
      out_shape=jax.ShapeDtypeStruct((G, 1), jnp.float32))(
          pm, Wc1, bc1, Wc2, bc2, Wo, bo)


def kernel(x, edge_index, batch, W1, b1, W2, b2, Wc1, bc1, Wc2, bc2, Wo, bo):
  src = edge_index[0]
  dst = edge_index[1]
  padidx = N + (jnp.arange(EP - E, dtype=jnp.int32) % (NP - N))
  src_p = jnp.concatenate([src, padidx]).reshape(NWIN, WIN)
  dst_p = jnp.concatenate([dst, padidx]).reshape(NWIN, WIN)

  degp = _sc_deg(dst_p)

  xp = jnp.pad(x, ((0, NP - N), (0, 0)))
  a1, t1, dinv = _tc_prep(xp, W1, degp)
  zrows = jnp.zeros((NZCH, 16), jnp.float32)
  s1 = _sc_agg(t1, src_p, dst_p, zrows)

  a2, t2 = _tc_mid(s1, a1, dinv, b1, W2)
  s2 = _sc_agg(t2, src_p, dst_p, zrows)

  h2 = _tc_post(s2, a2, dinv, b2)
  bp = jnp.concatenate([batch, jnp.full((NP - N,), G, jnp.int32)])
  pm = _sc_segmax(h2.reshape(NP * DH), bp).reshape(NW, GP, DH)
  return _tc_head(pm, Wc1, bc1, Wc2, bc2, Wo, bo)


# speedup vs baseline: 1.0675x; 1.0675x over previous
import functools

import jax
import jax.numpy as jnp
from jax import lax
from jax.experimental import pallas as pl
from jax.experimental.pallas import tpu as pltpu
from jax.experimental.pallas import tpu_sc as plsc

N = 100000
NP = 102400
E = 3200000
G = 1000
GP = 1024
DH = 64
NC, NS = 2, 16
NW = NC * NS
WIN = 128
J = 4
NWIN = 25088
EP = NWIN * WIN
ROWB = 4096
NSUB = NP // NW
NZCH = NP // NS

_mesh = functools.partial(
    plsc.VectorSubcoreMesh,
    core_axis_name="c", subcore_axis_name="s", num_cores=NC, num_subcores=NS)

_SC_PARAMS = pltpu.CompilerParams(use_tc_tiling_on_sc=False)


def _sc_deg(dst2d):
  @functools.partial(
      pl.kernel,
      out_type=jax.ShapeDtypeStruct((2 * NP,), jnp.float32),
      mesh=_mesh(),
      compiler_params=_SC_PARAMS,
      scratch_types=[
          pltpu.VMEM_SHARED((NP,), jnp.float32),
          pltpu.VMEM((2, J, WIN), jnp.int32),
          pltpu.VMEM((WIN,), jnp.float32),
          pltpu.VMEM((NZCH,), jnp.float32),
          pltpu.SemaphoreType.DMA,
      ])
  def k(dst_hbm, out_hbm, acc, iv, ones, zb, sem_i):
    c = lax.axis_index("c")
    s = lax.axis_index("s")

    @pl.loop(0, NZCH // 16)
    def _(i):
      zb[pl.ds(i * 16, 16)] = jnp.zeros((16,), jnp.float32)

    @pl.loop(0, WIN // 16)
    def _(i):
      ones[pl.ds(i * 16, 16)] = jnp.ones((16,), jnp.float32)

    pltpu.sync_copy(zb, acc.at[pl.ds(s * NZCH, NZCH)])
    plsc.subcore_barrier()

    per_sub = NWIN // 2 // NS
    base = (c * NS + s) * per_sub
    nb = per_sub // J

    def d_start(b, batch):
      pltpu.async_copy(dst_hbm.at[pl.ds(base + batch * J, J)], iv.at[b],
                       sem_i)

    def d_wait(b):
      pltpu.make_async_copy(dst_hbm.at[pl.ds(base, J)], iv.at[b],
                            sem_i).wait()

    d_start(0, 0)

    @pl.loop(0, nb // 2)
    def _(tp):
      for b in range(2):
        batch = 2 * tp + b
        d_wait(b)
        if b == 0:
          d_start(1, batch + 1)
        else:
          @pl.when(tp < nb // 2 - 1)
          def _():
            d_start(0, batch + 1)
        for j in range(J):
          pltpu.sync_copy(ones, acc.at[iv.at[b].at[j]], add=True)

    plsc.subcore_barrier()
    pltpu.sync_copy(acc.at[pl.ds(s * NZCH, NZCH)],
                    out_hbm.at[pl.ds(c * NP + s * NZCH, NZCH)])

  return k(dst2d)


def _sc_agg(t0, t1, t2, t3, src2d, dst2d, zrows):
  ot = [jax.ShapeDtypeStruct((NP, 16), jnp.float32)] * 4

  @functools.partial(
      pl.kernel,
      out_type=ot,
      mesh=_mesh(),
      compiler_params=_SC_PARAMS,
      scratch_types=[
          pltpu.VMEM_SHARED((NP, 16), jnp.float32),
          pltpu.VMEM((2, J, WIN), jnp.int32),
          pltpu.VMEM((4, J, WIN), jnp.int32),
          pltpu.VMEM((2, J, WIN, 16), jnp.float32),
          pltpu.SemaphoreType.DMA,
          pltpu.SemaphoreType.DMA,
          pltpu.SemaphoreType.DMA,
      ])
  def k(r0, r1, r2, r3, src_h, dst_h, zr, o0, o1, o2, o3,
        acc, siv, div, rows, sem_i, sem_g, sem_s):
    c = lax.axis_index("c")
    s = lax.axis_index("s")

    per_sub = NWIN // NS
    nbatch = per_sub // J
    nq = nbatch // 4

    def process(table, out):
      pltpu.sync_copy(zr, acc.at[pl.ds(s * NZCH, NZCH)])
      plsc.subcore_barrier()

      base = s * per_sub

      def start_idx(bs, bd, batch):
        w0 = base + batch * J
        pltpu.async_copy(src_h.at[pl.ds(w0, J)], siv.at[bs], sem_i)
        pltpu.async_copy(dst_h.at[pl.ds(w0, J)], div.at[bd], sem_i)

      def wait_idx(bs, bd):
        pltpu.make_async_copy(src_h.at[pl.ds(base, J)], siv.at[bs],
                              sem_i).wait()
        pltpu.make_async_copy(dst_h.at[pl.ds(base, J)], div.at[bd],
                              sem_i).wait()

      def drain_scatters(br, bd):
        for j in range(J):
          pltpu.make_async_copy(rows.at[br].at[j],
                                acc.at[div.at[bd].at[j]], sem_s).wait()

      def phase(q, tq):
        batch = 4 * tq + q
        br = q % 2
        if q >= 2:
          drain_scatters(br, q - 2)
        else:
          @pl.when(tq > 0)
          def _():
            drain_scatters(br, q + 2)
        wait_idx(br, q)
        nxt_s, nxt_d = (q + 1) % 2, (q + 1) % 4
        if q < 3:
          start_idx(nxt_s, nxt_d, batch + 1)
        else:
          @pl.when(tq < nq - 1)
          def _():
            start_idx(nxt_s, nxt_d, batch + 1)
        descs = [pltpu.async_copy(table.at[siv.at[br].at[j]],
                                  rows.at[br].at[j], sem_g)
                 for j in range(J)]
        for d in descs:
          d.wait()
        for j in range(J):
          pltpu.async_copy(rows.at[br].at[j], acc.at[div.at[q].at[j]],
                           sem_s, add=True)

      start_idx(0, 0, 0)

      @pl.loop(0, nq)
      def _(tq):
        for q in range(4):
          phase(q, tq)

      drain_scatters(0, 2)
      drain_scatters(1, 3)

      plsc.subcore_barrier()
      pltpu.sync_copy(acc.at[pl.ds(s * NZCH, NZCH)],
                      out.at[pl.ds(s * NZCH, NZCH)])
      plsc.subcore_barrier()

    @pl.when(c == 0)
    def _():
      process(r0, o0)
      process(r1, o1)

    @pl.when(c == 1)
    def _():
      process(r2, o2)
      process(r3, o3)

  return k(t0, t1, t2, t3, src2d, dst2d, zrows)


def _sc_segmax(h2p, batchp):
  @functools.partial(
      pl.kernel,
      out_type=jax.ShapeDtypeStruct((NW, GP * DH), jnp.float32),
      mesh=_mesh(),
      compiler_params=_SC_PARAMS,
      scratch_types=[
          pltpu.VMEM((GP * DH,), jnp.float32),
          pltpu.VMEM((320 * DH,), jnp.float32),
          pltpu.VMEM((NSUB,), jnp.int32),
      ])
  def k(h_h, b_h, o_h, accv, hv, bv):
    c = lax.axis_index("c")
    s = lax.axis_index("s")
    wid = s * NC + c

    @pl.loop(0, GP * DH // 16)
    def _(i):
      accv[pl.ds(i * 16, 16)] = jnp.full((16,), -jnp.inf, jnp.float32)

    pltpu.sync_copy(b_h.at[pl.ds(wid * NSUB, NSUB)], bv)

    ninf = jnp.full((16,), -jnp.inf, jnp.float32)

    def outer(t, carry):
      pltpu.sync_copy(h_h.at[pl.ds((wid * NSUB + t * 320) * DH, 320 * DH)],
                      hv)

      def inner(gi, cy):
        bvec = bv[pl.ds(t * 320 + gi * 16, 16)]
        for rr in range(16):
          cur, m0, m1, m2, m3 = cy
          r = gi * 16 + rr
          b = bvec[rr]
          r0 = hv[pl.ds(r * DH, 16)]
          r1 = hv[pl.ds(r * DH + 16, 16)]
          r2 = hv[pl.ds(r * DH + 32, 16)]
          r3 = hv[pl.ds(r * DH + 48, 16)]
          ch = b != cur

          @pl.when(ch)
          def _():
            accv[pl.ds(cur * DH, 16)] = m0
            accv[pl.ds(cur * DH + 16, 16)] = m1
            accv[pl.ds(cur * DH + 32, 16)] = m2
            accv[pl.ds(cur * DH + 48, 16)] = m3

          m0n = jnp.where(ch, r0, jnp.maximum(m0, r0))
          m1n = jnp.where(ch, r1, jnp.maximum(m1, r1))
          m2n = jnp.where(ch, r2, jnp.maximum(m2, r2))
          m3n = jnp.where(ch, r3, jnp.maximum(m3, r3))
          cy = (jnp.where(ch, b, cur), m0n, m1n, m2n, m3n)
        return cy

      return lax.fori_loop(0, 20, inner, carry)

    cur0 = bv[pl.ds(0, 16)][0]
    cur, m0, m1, m2, m3 = lax.fori_loop(
        0, NSUB // 320, outer, (cur0, ninf, ninf, ninf, ninf))
    accv[pl.ds(cur * DH, 16)] = m0
    accv[pl.ds(cur * DH + 16, 16)] = m1
    accv[pl.ds(cur * DH + 32, 16)] = m2
    accv[pl.ds(cur * DH + 48, 16)] = m3
    pltpu.sync_copy(accv, o_h.at[wid])

  return k(h2p, batchp)


def _tc_prep(xp, W1, degp):
  nb = NP // ROWB

  def body(xr, wr, d0r, d1r, o, o0, o1, o2, o3, od):
    dv1 = lax.rsqrt(d0r[...] + d1r[...] + 1.0)
    od[...] = dv1
    h = jnp.dot(xr[...], wr[...], preferred_element_type=jnp.float32)
    a = h * dv1[:, None]
    o[...] = a
    for p, op in enumerate((o0, o1, o2, o3)):
      op[...] = a[:, p * 16:(p + 1) * 16]

  ts = jax.ShapeDtypeStruct((NP, 16), jnp.float32)
  return pl.pallas_call(
      body, grid=(nb,),
      in_specs=[pl.BlockSpec((ROWB, 16), lambda i: (i, 0)),
                pl.BlockSpec((16, DH), lambda i: (0, 0)),
                pl.BlockSpec((ROWB,), lambda i: (i,)),
                pl.BlockSpec((ROWB,), lambda i: (i + nb,))],
      out_specs=[pl.BlockSpec((ROWB, DH), lambda i: (i, 0))] +
                [pl.BlockSpec((ROWB, 16), lambda i: (i, 0))] * 4 +
                [pl.BlockSpec((ROWB,), lambda i: (i,))],
      out_shape=[jax.ShapeDtypeStruct((NP, DH), jnp.float32),
                 ts, ts, ts, ts,
                 jax.ShapeDtypeStruct((NP,), jnp.float32)])(
          xp, W1, degp, degp)


def _tc_mid(s0, s1, s2, s3, a, dinv, b, W):
  def body(s0r, s1r, s2r, s3r, ar, dr, br, wr, o, o0, o1, o2, o3):
    sf = jnp.concatenate([s0r[...], s1r[...], s2r[...], s3r[...]], axis=1)
    dv = dr[...][:, None]
    x2 = jnp.maximum((sf + ar[...]) * dv + br[...][None, :], 0.0)
    a2 = jnp.dot(x2, wr[...], preferred_element_type=jnp.float32) * dv
    o[...] = a2
    for p, op in enumerate((o0, o1, o2, o3)):
      op[...] = a2[:, p * 16:(p + 1) * 16]

  ts = jax.ShapeDtypeStruct((NP, 16), jnp.float32)
  return pl.pallas_call(
      body, grid=(NP // ROWB,),
      in_specs=[pl.BlockSpec((ROWB, 16), lambda i: (i, 0))] * 4 +
               [pl.BlockSpec((ROWB, DH), lambda i: (i, 0)),
                pl.BlockSpec((ROWB,), lambda i: (i,)),
                pl.BlockSpec((DH,), lambda i: (0,)),
                pl.BlockSpec((DH, DH), lambda i: (0, 0))],
      out_specs=[pl.BlockSpec((ROWB, DH), lambda i: (i, 0))] +
                [pl.BlockSpec((ROWB, 16), lambda i: (i, 0))] * 4,
      out_shape=[jax.ShapeDtypeStruct((NP, DH), jnp.float32),
                 ts, ts, ts, ts])(
          s0, s1, s2, s3, a, dinv, b, W)


def _tc_post(s0, s1, s2, s3, a, dinv, b):
  def body(s0r, s1r, s2r, s3r, ar, dr, br, o):
    sf = jnp.concatenate([s0r[...], s1r[...], s2r[...], s3r[...]], axis=1)
    dv = dr[...][:, None]
    o[...] = jnp.maximum((sf + ar[...]) * dv + br[...][None, :], 0.0)

  return pl.pallas_call(
      body, grid=(NP // ROWB,),
      in_specs=[pl.BlockSpec((ROWB, 16), lambda i: (i, 0))] * 4 +
               [pl.BlockSpec((ROWB, DH), lambda i: (i, 0)),
                pl.BlockSpec((ROWB,), lambda i: (i,)),
                pl.BlockSpec((DH,), lambda i: (0,))],
      out_specs=pl.BlockSpec((ROWB, DH), lambda i: (i, 0)),
      out_shape=jax.ShapeDtypeStruct((NP, DH), jnp.float32))(
          s0, s1, s2, s3, a, dinv, b)


def _tc_head(pm, Wc1, bc1, Wc2, bc2, Wo, bo):
  def body(pr, w1r, b1r, w2r, b2r, wor, bor, o):
    g = jnp.max(pr[...], axis=0)[:G]
    g = jnp.maximum(
        jnp.dot(g, w1r[...], preferred_element_type=jnp.float32)
        + b1r[...][None, :], 0.0)
    g = jnp.maximum(
        jnp.dot(g, w2r[...], preferred_element_type=jnp.float32)
        + b2r[...][None, :], 0.0)
    z = (jnp.dot(g, wor[...], preferred_element_type=jnp.float32)
         + bor[...][None, :])
    o[...] = 1.0 / (1.0 + jnp.exp(-z))

  return pl.pallas_call(
      body,
      in_specs=[pl.BlockSpec((NW, GP, DH), lambda: (0, 0, 0)),
                pl.BlockSpec((DH, DH), lambda: (0, 0)),
                pl.BlockSpec((DH,), lambda: (0,)),
                pl.BlockSpec((DH, DH), lambda: (0, 0)),
                pl.BlockSpec((DH,), lambda: (0,)),
                pl.BlockSpec((DH, 1), lambda: (0, 0)),
                pl.BlockSpec((1,), lambda: (0,))],
      out_specs=pl.BlockSpec((G, 1), lambda: (0, 0)),
      out_shape=jax.ShapeDtypeStruct((G, 1), jnp.float32))(
          pm, Wc1, bc1, Wc2, bc2, Wo, bo)


def kernel(x, edge_index, batch, W1, b1, W2, b2, Wc1, bc1, Wc2, bc2, Wo, bo):
  src = edge_index[0]
  dst = edge_index[1]
  padidx = N + (jnp.arange(EP - E, dtype=jnp.int32) % (NP - N))
  src_p = jnp.concatenate([src, padidx]).reshape(NWIN, WIN)
  dst_p = jnp.concatenate([dst, padidx]).reshape(NWIN, WIN)

  degp = _sc_deg(dst_p)

  xp = jnp.pad(x, ((0, NP - N), (0, 0)))
  a1, t10, t11, t12, t13, dinv = _tc_prep(xp, W1, degp)
  zrows = jnp.zeros((NZCH, 16), jnp.float32)
  s1 = _sc_agg(t10, t11, t12, t13, src_p, dst_p, zrows)

  a2, t20, t21, t22, t23 = _tc_mid(*s1, a1, dinv, b1, W2)
  s2 = _sc_agg(t20, t21, t22, t23, src_p, dst_p, zrows)

  h2 = _tc_post(*s2, a2, dinv, b2)
  bp = jnp.concatenate([batch, jnp.full((NP - N,), G, jnp.int32)])
  pm = _sc_segmax(h2.reshape(NP * DH), bp).reshape(NW, GP, DH)
  return _tc_head(pm, Wc1, bc1, Wc2, bc2, Wo, bo)
